# Initial kernel scaffold; baseline (speedup 1.0000x reference)
import jax
import jax.numpy as jnp
from jax.experimental import pallas as pl


def kernel(edge_index, user_n_j, item_n_j, user_emb, item_emb, mean_weight):
    raise NotImplementedError("write your pallas kernel here")


# speedup vs baseline: 11.7525x; 11.7525x over previous
import functools

import jax
import jax.numpy as jnp
from jax import lax
from jax.experimental import pallas as pl
from jax.experimental.pallas import tpu as pltpu
from jax.experimental.pallas import tpu_sc as plsc

N_USERS = 100000
N_ITEMS = 100000
EMBED = 32
HALF = 16

CH = 128
G = 8
CHUNKS_PER_TILE = 784
N_CHUNKS = CHUNKS_PER_TILE * 16
E_PAD = N_CHUNKS * CH
ROWS_PER_TILE = 6272
ACC_ROWS = ROWS_PER_TILE * 16


def _sc_segment_sum(rows2d, cols2d, emb_lo, emb_hi):
    mesh = plsc.VectorSubcoreMesh(core_axis_name="c", subcore_axis_name="s")

    @functools.partial(
        pl.kernel,
        out_type=(
            jax.ShapeDtypeStruct((N_USERS, HALF), jnp.float32),
            jax.ShapeDtypeStruct((N_USERS, HALF), jnp.float32),
        ),
        mesh=mesh,
        scratch_types=[
            pltpu.VMEM((G, CH), jnp.int32),
            pltpu.VMEM((G, CH), jnp.int32),
            pltpu.VMEM((G, CH, HALF), jnp.float32),
            pltpu.VMEM((CH, HALF), jnp.float32),
            pltpu.VMEM_SHARED((ACC_ROWS, HALF), jnp.float32),
            pltpu.SemaphoreType.DMA,
            pltpu.SemaphoreType.DMA,
        ],
        compiler_params=pltpu.CompilerParams(use_tc_tiling_on_sc=False),
    )
    def seg(rows_hbm, cols_hbm, emb_lo_hbm, emb_hi_hbm, out_lo, out_hi,
            rowsb, colsb, gath, zbuf, acc, gsem, ssem):
        c = lax.axis_index("c")
        s = lax.axis_index("s")

        def zb(i, carry):
            zbuf[i, :] = jnp.zeros((HALF,), jnp.float32)
            return carry

        lax.fori_loop(0, CH, zb, 0)

        def za(k, carry):
            pltpu.sync_copy(zbuf, acc.at[pl.ds(s * ROWS_PER_TILE + k * CH, CH)])
            return carry

        lax.fori_loop(0, ROWS_PER_TILE // CH, za, 0)
        plsc.subcore_barrier()

        def run(emb, out):
            def grp(g, carry):
                base = s * CHUNKS_PER_TILE + g * G
                pltpu.sync_copy(rows_hbm.at[pl.ds(base, G)], rowsb)
                pltpu.sync_copy(cols_hbm.at[pl.ds(base, G)], colsb)
                gets = [
                    pltpu.async_copy(emb.at[colsb.at[j]], gath.at[j], gsem)
                    for j in range(G)
                ]
                for d in gets:
                    d.wait()
                puts = [
                    pltpu.async_copy(gath.at[j], acc.at[rowsb.at[j]], ssem,
                                     add=True)
                    for j in range(G)
                ]
                for d in puts:
                    d.wait()
                return carry

            lax.fori_loop(0, CHUNKS_PER_TILE // G, grp, 0)
            plsc.subcore_barrier()

            @pl.when(s < 15)
            def _():
                off = s * ROWS_PER_TILE
                pltpu.sync_copy(acc.at[pl.ds(off, ROWS_PER_TILE)],
                                out.at[pl.ds(off, ROWS_PER_TILE)])

            @pl.when(s == 15)
            def _():
                off = 15 * ROWS_PER_TILE
                rem = N_USERS - off
                pltpu.sync_copy(acc.at[pl.ds(off, rem)],
                                out.at[pl.ds(off, rem)])

        @pl.when(c == 0)
        def _():
            run(emb_lo_hbm, out_lo)

        @pl.when(c == 1)
        def _():
            run(emb_hi_hbm, out_hi)

    return seg(rows2d, cols2d, emb_lo, emb_hi)


def _tc_finish_body(e_lo_ref, e_hi_ref, nj_ref, w_lo_ref, w_hi_ref, out_ref):
    acc = jnp.dot(e_lo_ref[...], w_lo_ref[...],
                  preferred_element_type=jnp.float32)
    acc += jnp.dot(e_hi_ref[...], w_hi_ref[...],
                   preferred_element_type=jnp.float32)
    out_ref[...] = acc * nj_ref[...]


def _tc_finish(e_lo, e_hi, user_n_j, mean_weight):
    blk = 5000
    grid = (N_USERS // blk,)
    w_lo = mean_weight[:HALF, :]
    w_hi = mean_weight[HALF:, :]
    return pl.pallas_call(
        _tc_finish_body,
        grid=grid,
        in_specs=[
            pl.BlockSpec((blk, HALF), lambda i: (i, 0)),
            pl.BlockSpec((blk, HALF), lambda i: (i, 0)),
            pl.BlockSpec((blk, 1), lambda i: (i, 0)),
            pl.BlockSpec((HALF, EMBED), lambda i: (0, 0)),
            pl.BlockSpec((HALF, EMBED), lambda i: (0, 0)),
        ],
        out_specs=pl.BlockSpec((blk, EMBED), lambda i: (i, 0)),
        out_shape=jax.ShapeDtypeStruct((N_USERS, EMBED), jnp.float32),
    )(e_lo, e_hi, user_n_j, w_lo, w_hi)


def kernel(edge_index, user_n_j, item_n_j, user_emb, item_emb, mean_weight):
    rows = edge_index[0].astype(jnp.int32)
    cols = edge_index[1].astype(jnp.int32)
    n_pad = E_PAD - rows.shape[0]
    rows_p = jnp.concatenate(
        [rows, jnp.full((n_pad,), N_USERS, jnp.int32)]).reshape(N_CHUNKS, CH)
    cols_p = jnp.concatenate(
        [cols, jnp.zeros((n_pad,), jnp.int32)]).reshape(N_CHUNKS, CH)
    emb_lo = item_emb[:, :HALF]
    emb_hi = item_emb[:, HALF:]
    e_lo, e_hi = _sc_segment_sum(rows_p, cols_p, emb_lo, emb_hi)
    return _tc_finish(e_lo, e_hi, user_n_j, mean_weight)


# speedup vs baseline: 15.6322x; 1.3301x over previous
import functools

import jax
import jax.numpy as jnp
from jax import lax
from jax.experimental import pallas as pl
from jax.experimental.pallas import tpu as pltpu
from jax.experimental.pallas import tpu_sc as plsc

N_USERS = 100000
N_ITEMS = 100000
EMBED = 32
HALF = 16

CH = 128
G = 2
NSLOT = 6
AG = 3
AI = NSLOT - 1
ZR = 64
CHUNKS_PER_TILE = 784
N_GROUPS = CHUNKS_PER_TILE // G
N_CHUNKS = CHUNKS_PER_TILE * 16
E_PAD = N_CHUNKS * CH
ROWS_PER_TILE = 6272
ACC_ROWS = ROWS_PER_TILE * 16


def _sc_segment_sum(edges_il, emb_lo, emb_hi):
    mesh = plsc.VectorSubcoreMesh(core_axis_name="c", subcore_axis_name="s")

    @functools.partial(
        pl.kernel,
        out_type=(
            jax.ShapeDtypeStruct((N_USERS, HALF), jnp.float32),
            jax.ShapeDtypeStruct((N_USERS, HALF), jnp.float32),
        ),
        mesh=mesh,
        scratch_types=[
            pltpu.VMEM((NSLOT, G, 2, CH), jnp.int32),
            pltpu.VMEM((NSLOT, G, CH, HALF), jnp.float32),
            pltpu.VMEM((ZR, HALF), jnp.float32),
            pltpu.VMEM_SHARED((ACC_ROWS, HALF), jnp.float32),
            pltpu.SemaphoreType.DMA((NSLOT,)),
            pltpu.SemaphoreType.DMA((NSLOT,)),
            pltpu.SemaphoreType.DMA((NSLOT,)),
        ],
        compiler_params=pltpu.CompilerParams(use_tc_tiling_on_sc=False),
    )
    def seg(edges_hbm, emb_lo_hbm, emb_hi_hbm, out_lo, out_hi,
            idxb, gath, zbuf, acc, isem, gsem, ssem):
        c = lax.axis_index("c")
        s = lax.axis_index("s")

        def idx_desc(slot, grp):
            base = s * CHUNKS_PER_TILE + grp * G
            return pltpu.make_async_copy(
                edges_hbm.at[pl.ds(base, G)], idxb.at[slot], isem.at[slot])

        def fire_gathers(emb, slot):
            for j in range(G):
                pltpu.async_copy(emb.at[idxb.at[slot, j, 1]],
                                 gath.at[slot, j], gsem.at[slot])

        def drain_gathers(emb, slot):
            for j in range(G):
                pltpu.make_async_copy(emb.at[idxb.at[slot, j, 1]],
                                      gath.at[slot, j], gsem.at[slot]).wait()

        def fire_scatters(slot):
            for j in range(G):
                pltpu.async_copy(gath.at[slot, j], acc.at[idxb.at[slot, j, 0]],
                                 ssem.at[slot], add=True)

        def drain_scatters(slot):
            for j in range(G):
                pltpu.make_async_copy(gath.at[slot, j],
                                      acc.at[idxb.at[slot, j, 0]],
                                      ssem.at[slot]).wait()

        def run(emb, out):
            for q in range(NSLOT):
                idx_desc(q, q).start()
            for q in range(AG):
                idx_desc(q, q).wait()
                fire_gathers(emb, q)

            def zb(i, carry):
                zbuf[i, :] = jnp.zeros((HALF,), jnp.float32)
                return carry

            lax.fori_loop(0, ZR, zb, 0)

            def za(k, carry):
                pltpu.sync_copy(
                    zbuf, acc.at[pl.ds(s * ROWS_PER_TILE + k * ZR, ZR)])
                return carry

            lax.fori_loop(0, ROWS_PER_TILE // ZR, za, 0)
            plsc.subcore_barrier()

            def grp(r, carry):
                p = lax.rem(r, NSLOT)
                pg = lax.rem(r + AG, NSLOT)
                pi = lax.rem(r + AI, NSLOT)
                drain_gathers(emb, p)
                fire_scatters(p)

                @pl.when(r + AG < N_GROUPS)
                def _():
                    idx_desc(pg, r + AG).wait()
                    fire_gathers(emb, pg)

                @pl.when(jnp.logical_and(r >= 1, r + AI < N_GROUPS))
                def _():
                    drain_scatters(pi)
                    idx_desc(pi, r + AI).start()

                return carry

            lax.fori_loop(0, N_GROUPS, grp, 0)
            for g in range(N_GROUPS - NSLOT, N_GROUPS):
                drain_scatters(g % NSLOT)
            plsc.subcore_barrier()

            @pl.when(s < 15)
            def _():
                off = s * ROWS_PER_TILE
                pltpu.sync_copy(acc.at[pl.ds(off, ROWS_PER_TILE)],
                                out.at[pl.ds(off, ROWS_PER_TILE)])

            @pl.when(s == 15)
            def _():
                off = 15 * ROWS_PER_TILE
                rem = N_USERS - off
                pltpu.sync_copy(acc.at[pl.ds(off, rem)],
                                out.at[pl.ds(off, rem)])

        @pl.when(c == 0)
        def _():
            run(emb_lo_hbm, out_lo)

        @pl.when(c == 1)
        def _():
            run(emb_hi_hbm, out_hi)

    return seg(edges_il, emb_lo, emb_hi)


def _tc_finish_body(e_lo_ref, e_hi_ref, nj_ref, w_lo_ref, w_hi_ref, out_ref):
    acc = jnp.dot(e_lo_ref[...], w_lo_ref[...],
                  preferred_element_type=jnp.float32)
    acc += jnp.dot(e_hi_ref[...], w_hi_ref[...],
                   preferred_element_type=jnp.float32)
    out_ref[...] = acc * nj_ref[...]


def _tc_finish(e_lo, e_hi, user_n_j, mean_weight):
    blk = 5000
    grid = (N_USERS // blk,)
    w_lo = mean_weight[:HALF, :]
    w_hi = mean_weight[HALF:, :]
    return pl.pallas_call(
        _tc_finish_body,
        grid=grid,
        in_specs=[
            pl.BlockSpec((blk, HALF), lambda i: (i, 0)),
            pl.BlockSpec((blk, HALF), lambda i: (i, 0)),
            pl.BlockSpec((blk, 1), lambda i: (i, 0)),
            pl.BlockSpec((HALF, EMBED), lambda i: (0, 0)),
            pl.BlockSpec((HALF, EMBED), lambda i: (0, 0)),
        ],
        out_specs=pl.BlockSpec((blk, EMBED), lambda i: (i, 0)),
        out_shape=jax.ShapeDtypeStruct((N_USERS, EMBED), jnp.float32),
    )(e_lo, e_hi, user_n_j, w_lo, w_hi)


def kernel(edge_index, user_n_j, item_n_j, user_emb, item_emb, mean_weight):
    rows = edge_index[0].astype(jnp.int32)
    cols = edge_index[1].astype(jnp.int32)
    n_pad = E_PAD - rows.shape[0]
    rows_p = jnp.concatenate(
        [rows, jnp.full((n_pad,), N_USERS, jnp.int32)]).reshape(N_CHUNKS, CH)
    cols_p = jnp.concatenate(
        [cols, jnp.zeros((n_pad,), jnp.int32)]).reshape(N_CHUNKS, CH)
    edges_il = jnp.stack([rows_p, cols_p], axis=1)
    emb_lo = item_emb[:, :HALF]
    emb_hi = item_emb[:, HALF:]
    e_lo, e_hi = _sc_segment_sum(edges_il, emb_lo, emb_hi)
    return _tc_finish(e_lo, e_hi, user_n_j, mean_weight)


# speedup vs baseline: 20.8947x; 1.3366x over previous
import functools

import jax
import jax.numpy as jnp
from jax import lax
from jax.experimental import pallas as pl
from jax.experimental.pallas import tpu as pltpu
from jax.experimental.pallas import tpu_sc as plsc

N_USERS = 100000
N_ITEMS = 100000
EMBED = 32
HALF = 16

CH = 128
G = 2
NSLOT = 6
AG = 3
AI = NSLOT - 1
ZR = 64
N_EDGE = 1600000
N_CHUNKS = N_EDGE // CH
CHUNKS_PER_TILE = N_CHUNKS // 16
N_MAIN = CHUNKS_PER_TILE - 1
N_GROUPS = N_MAIN // G
ROWS_PER_TILE = 6272
ACC_ROWS = ROWS_PER_TILE * 16
U_PAD = 100032
PACK_ROWS = U_PAD // 8


def _sc_segment_sum(rows1d, cols1d, emb_lo, emb_hi):
    mesh = plsc.VectorSubcoreMesh(core_axis_name="c", subcore_axis_name="s")

    @functools.partial(
        pl.kernel,
        out_type=(
            jax.ShapeDtypeStruct((U_PAD, HALF), jnp.float32),
            jax.ShapeDtypeStruct((U_PAD, HALF), jnp.float32),
        ),
        mesh=mesh,
        scratch_types=[
            pltpu.VMEM((NSLOT, 2, G, CH), jnp.int32),
            pltpu.VMEM((NSLOT, G, CH, HALF), jnp.float32),
            pltpu.VMEM((ZR, HALF), jnp.float32),
            pltpu.VMEM_SHARED((ACC_ROWS, HALF), jnp.float32),
            pltpu.SemaphoreType.DMA((NSLOT,)),
            pltpu.SemaphoreType.DMA((NSLOT,)),
            pltpu.SemaphoreType.DMA((NSLOT,)),
        ],
        compiler_params=pltpu.CompilerParams(use_tc_tiling_on_sc=False),
    )
    def seg(rows_hbm, cols_hbm, emb_lo_hbm, emb_hi_hbm, out_lo, out_hi,
            idxb, gath, zbuf, acc, isem, gsem, ssem):
        c = lax.axis_index("c")
        s = lax.axis_index("s")

        def idx_descs(slot, chunk0):
            ds = []
            for j in range(G):
                base = (chunk0 + j) * CH
                ds.append(pltpu.make_async_copy(
                    rows_hbm.at[pl.ds(base, CH)], idxb.at[slot, 0, j],
                    isem.at[slot]))
                ds.append(pltpu.make_async_copy(
                    cols_hbm.at[pl.ds(base, CH)], idxb.at[slot, 1, j],
                    isem.at[slot]))
            return ds

        def fire_gathers(emb, slot):
            for j in range(G):
                pltpu.async_copy(emb.at[idxb.at[slot, 1, j]],
                                 gath.at[slot, j], gsem.at[slot])

        def drain_gathers(emb, slot):
            for j in range(G):
                pltpu.make_async_copy(emb.at[idxb.at[slot, 1, j]],
                                      gath.at[slot, j], gsem.at[slot]).wait()

        def fire_scatters(slot):
            for j in range(G):
                pltpu.async_copy(gath.at[slot, j], acc.at[idxb.at[slot, 0, j]],
                                 ssem.at[slot], add=True)

        def drain_scatters(slot):
            for j in range(G):
                pltpu.make_async_copy(gath.at[slot, j],
                                      acc.at[idxb.at[slot, 0, j]],
                                      ssem.at[slot]).wait()

        def run(emb, out):
            chunk_base = s * CHUNKS_PER_TILE
            for q in range(NSLOT):
                for d in idx_descs(q, chunk_base + q * G):
                    d.start()
            for q in range(AG):
                for d in idx_descs(q, chunk_base + q * G):
                    d.wait()
                fire_gathers(emb, q)

            def zb(i, carry):
                zbuf[i, :] = jnp.zeros((HALF,), jnp.float32)
                return carry

            lax.fori_loop(0, ZR, zb, 0)

            def za(k, carry):
                pltpu.sync_copy(
                    zbuf, acc.at[pl.ds(s * ROWS_PER_TILE + k * ZR, ZR)])
                return carry

            lax.fori_loop(0, ROWS_PER_TILE // ZR, za, 0)
            plsc.subcore_barrier()

            def grp(r, carry):
                p = lax.rem(r, NSLOT)
                pg = lax.rem(r + AG, NSLOT)
                pi = lax.rem(r + AI, NSLOT)
                drain_gathers(emb, p)
                fire_scatters(p)

                @pl.when(r + AG < N_GROUPS)
                def _():
                    for d in idx_descs(pg, chunk_base + (r + AG) * G):
                        d.wait()
                    fire_gathers(emb, pg)

                @pl.when(jnp.logical_and(r >= 1, r + AI < N_GROUPS))
                def _():
                    drain_scatters(pi)
                    for d in idx_descs(pi, chunk_base + (r + AI) * G):
                        d.start()

                return carry

            lax.fori_loop(0, N_GROUPS, grp, 0)
            for g in range(N_GROUPS - NSLOT, N_GROUPS):
                drain_scatters(g % NSLOT)

            def do_chunk(chunk_idx):
                base = chunk_idx * CH
                pltpu.sync_copy(rows_hbm.at[pl.ds(base, CH)],
                                idxb.at[0, 0, 0])
                pltpu.sync_copy(cols_hbm.at[pl.ds(base, CH)],
                                idxb.at[0, 1, 0])
                pltpu.async_copy(emb.at[idxb.at[0, 1, 0]], gath.at[0, 0],
                                 gsem.at[0]).wait()
                pltpu.async_copy(gath.at[0, 0], acc.at[idxb.at[0, 0, 0]],
                                 ssem.at[0], add=True).wait()

            do_chunk(chunk_base + N_MAIN)

            @pl.when(s < 4)
            def _():
                do_chunk(16 * CHUNKS_PER_TILE + s)

            plsc.subcore_barrier()

            @pl.when(s < 15)
            def _():
                off = s * ROWS_PER_TILE
                pltpu.sync_copy(acc.at[pl.ds(off, ROWS_PER_TILE)],
                                out.at[pl.ds(off, ROWS_PER_TILE)])

            @pl.when(s == 15)
            def _():
                off = 15 * ROWS_PER_TILE
                rem = U_PAD - off
                pltpu.sync_copy(acc.at[pl.ds(off, rem)],
                                out.at[pl.ds(off, rem)])

        @pl.when(c == 0)
        def _():
            run(emb_lo_hbm, out_lo)

        @pl.when(c == 1)
        def _():
            run(emb_hi_hbm, out_hi)

    return seg(rows1d, cols1d, emb_lo, emb_hi)


def _tc_finish_body(ep_lo_ref, ep_hi_ref, njp_ref, wb_lo_ref, wb_hi_ref,
                    s_ref, out_ref):
    acc = jnp.dot(ep_lo_ref[...], wb_lo_ref[...],
                  preferred_element_type=jnp.float32)
    acc += jnp.dot(ep_hi_ref[...], wb_hi_ref[...],
                   preferred_element_type=jnp.float32)
    scale = jnp.dot(njp_ref[...], s_ref[...],
                    preferred_element_type=jnp.float32)
    out_ref[...] = acc * scale


def _tc_finish(ep_lo, ep_hi, njp, mean_weight):
    blk = PACK_ROWS // 3
    grid = (3,)
    eye8 = jnp.eye(8, dtype=jnp.float32)
    wb_lo = jnp.kron(eye8, mean_weight[:HALF, :])
    wb_hi = jnp.kron(eye8, mean_weight[HALF:, :])
    sel = jnp.kron(eye8, jnp.ones((1, EMBED), jnp.float32))
    return pl.pallas_call(
        _tc_finish_body,
        grid=grid,
        in_specs=[
            pl.BlockSpec((blk, 128), lambda i: (i, 0)),
            pl.BlockSpec((blk, 128), lambda i: (i, 0)),
            pl.BlockSpec((blk, 8), lambda i: (i, 0)),
            pl.BlockSpec((128, 256), lambda i: (0, 0)),
            pl.BlockSpec((128, 256), lambda i: (0, 0)),
            pl.BlockSpec((8, 256), lambda i: (0, 0)),
        ],
        out_specs=pl.BlockSpec((blk, 256), lambda i: (i, 0)),
        out_shape=jax.ShapeDtypeStruct((PACK_ROWS, 256), jnp.float32),
    )(ep_lo, ep_hi, njp, wb_lo, wb_hi, sel)


def kernel(edge_index, user_n_j, item_n_j, user_emb, item_emb, mean_weight):
    rows1d = edge_index[0].astype(jnp.int32)
    cols1d = edge_index[1].astype(jnp.int32)
    emb_lo = item_emb[:, :HALF]
    emb_hi = item_emb[:, HALF:]
    e_lo, e_hi = _sc_segment_sum(rows1d, cols1d, emb_lo, emb_hi)
    ep_lo = e_lo.reshape(PACK_ROWS, 128)
    ep_hi = e_hi.reshape(PACK_ROWS, 128)
    njp = jnp.pad(user_n_j[:, 0], (0, U_PAD - N_USERS)).reshape(PACK_ROWS, 8)
    out_pack = _tc_finish(ep_lo, ep_hi, njp, mean_weight)
    return out_pack.reshape(U_PAD, EMBED)[:N_USERS]


# speedup vs baseline: 24.7282x; 1.1835x over previous
import functools

import jax
import jax.numpy as jnp
from jax import lax
from jax.experimental import pallas as pl
from jax.experimental.pallas import tpu as pltpu
from jax.experimental.pallas import tpu_sc as plsc

N_USERS = 100000
N_ITEMS = 100000
EMBED = 32
HALF = 16

CH = 128
G = 2
NSLOT = 6
AG = 3
AI = NSLOT - 1
ZR = 64
N_EDGE = 1600000
N_CHUNKS = N_EDGE // CH
CHUNKS_PER_TILE = N_CHUNKS // 16
N_MAIN = CHUNKS_PER_TILE - 1
N_GROUPS = N_MAIN // G
ROWS_PER_TILE = 6272
ACC_ROWS = ROWS_PER_TILE * 16
PACK_ROWS = N_USERS // 8


def _sc_segment_sum(rows1d, cols1d, table):
    mesh = plsc.VectorSubcoreMesh(core_axis_name="c", subcore_axis_name="s")

    @functools.partial(
        pl.kernel,
        out_type=(
            jax.ShapeDtypeStruct((N_USERS, HALF), jnp.float32),
            jax.ShapeDtypeStruct((N_USERS, HALF), jnp.float32),
        ),
        mesh=mesh,
        scratch_types=[
            pltpu.VMEM((NSLOT, 2, G, CH), jnp.int32),
            pltpu.VMEM((NSLOT, G, CH, HALF), jnp.float32),
            pltpu.VMEM((ZR, HALF), jnp.float32),
            pltpu.VMEM_SHARED((ACC_ROWS, HALF), jnp.float32),
            pltpu.SemaphoreType.DMA((NSLOT,)),
            pltpu.SemaphoreType.DMA((NSLOT,)),
            pltpu.SemaphoreType.DMA((NSLOT,)),
        ],
        compiler_params=pltpu.CompilerParams(use_tc_tiling_on_sc=False),
    )
    def seg(rows_hbm, cols_hbm, table_hbm, out_lo, out_hi,
            idxb, gath, zbuf, acc, isem, gsem, ssem):
        c = lax.axis_index("c")
        s = lax.axis_index("s")

        def idx_descs(slot, chunk0):
            ds = []
            for j in range(G):
                base = (chunk0 + j) * CH
                ds.append(pltpu.make_async_copy(
                    rows_hbm.at[pl.ds(base, CH)], idxb.at[slot, 0, j],
                    isem.at[slot]))
                ds.append(pltpu.make_async_copy(
                    cols_hbm.at[pl.ds(base, CH)], idxb.at[slot, 1, j],
                    isem.at[slot]))
            return ds

        def transform_cols(slot):
            for j in range(G):
                for k in range(CH // 16):
                    sl = pl.ds(k * 16, 16)
                    v = idxb[slot, 1, j, sl]
                    idxb[slot, 1, j, sl] = v * 2 + c

        def fire_gathers(slot):
            for j in range(G):
                pltpu.async_copy(table_hbm.at[idxb.at[slot, 1, j]],
                                 gath.at[slot, j], gsem.at[slot])

        def drain_gathers(slot):
            for j in range(G):
                pltpu.make_async_copy(table_hbm.at[idxb.at[slot, 1, j]],
                                      gath.at[slot, j], gsem.at[slot]).wait()

        def fire_scatters(slot):
            for j in range(G):
                pltpu.async_copy(gath.at[slot, j], acc.at[idxb.at[slot, 0, j]],
                                 ssem.at[slot], add=True)

        def drain_scatters(slot):
            for j in range(G):
                pltpu.make_async_copy(gath.at[slot, j],
                                      acc.at[idxb.at[slot, 0, j]],
                                      ssem.at[slot]).wait()

        def run(out):
            chunk_base = s * CHUNKS_PER_TILE
            for q in range(NSLOT):
                for d in idx_descs(q, chunk_base + q * G):
                    d.start()
            for q in range(AG):
                for d in idx_descs(q, chunk_base + q * G):
                    d.wait()
                transform_cols(q)
                fire_gathers(q)

            def zb(i, carry):
                zbuf[i, :] = jnp.zeros((HALF,), jnp.float32)
                return carry

            lax.fori_loop(0, ZR, zb, 0)

            def za(k, carry):
                pltpu.sync_copy(
                    zbuf, acc.at[pl.ds(s * ROWS_PER_TILE + k * ZR, ZR)])
                return carry

            lax.fori_loop(0, ROWS_PER_TILE // ZR, za, 0)
            plsc.subcore_barrier()

            def grp(r, carry):
                p = lax.rem(r, NSLOT)
                pg = lax.rem(r + AG, NSLOT)
                pi = lax.rem(r + AI, NSLOT)
                drain_gathers(p)
                fire_scatters(p)

                @pl.when(r + AG < N_GROUPS)
                def _():
                    for d in idx_descs(pg, chunk_base + (r + AG) * G):
                        d.wait()
                    transform_cols(pg)
                    fire_gathers(pg)

                @pl.when(jnp.logical_and(r >= 1, r + AI < N_GROUPS))
                def _():
                    drain_scatters(pi)
                    for d in idx_descs(pi, chunk_base + (r + AI) * G):
                        d.start()

                return carry

            lax.fori_loop(0, N_GROUPS, grp, 0)
            for g in range(N_GROUPS - NSLOT, N_GROUPS):
                drain_scatters(g % NSLOT)

            def do_chunk(chunk_idx):
                base = chunk_idx * CH
                pltpu.sync_copy(rows_hbm.at[pl.ds(base, CH)],
                                idxb.at[0, 0, 0])
                pltpu.sync_copy(cols_hbm.at[pl.ds(base, CH)],
                                idxb.at[0, 1, 0])
                for k in range(CH // 16):
                    sl = pl.ds(k * 16, 16)
                    v = idxb[0, 1, 0, sl]
                    idxb[0, 1, 0, sl] = v * 2 + c
                pltpu.async_copy(table_hbm.at[idxb.at[0, 1, 0]],
                                 gath.at[0, 0], gsem.at[0]).wait()
                pltpu.async_copy(gath.at[0, 0], acc.at[idxb.at[0, 0, 0]],
                                 ssem.at[0], add=True).wait()

            do_chunk(chunk_base + N_MAIN)

            @pl.when(s < 4)
            def _():
                do_chunk(16 * CHUNKS_PER_TILE + s)

            plsc.subcore_barrier()

            @pl.when(s < 15)
            def _():
                off = s * ROWS_PER_TILE
                pltpu.sync_copy(acc.at[pl.ds(off, ROWS_PER_TILE)],
                                out.at[pl.ds(off, ROWS_PER_TILE)])

            @pl.when(s == 15)
            def _():
                off = 15 * ROWS_PER_TILE
                rem = N_USERS - off
                pltpu.sync_copy(acc.at[pl.ds(off, rem)],
                                out.at[pl.ds(off, rem)])

        @pl.when(c == 0)
        def _():
            run(out_lo)

        @pl.when(c == 1)
        def _():
            run(out_hi)

    return seg(rows1d, cols1d, table)


PBLK = 256
OBLK = PBLK * 8


def _tc_finish_body(ep_lo_ref, ep_hi_ref, njp_ref, wb_lo_ref, wb_hi_ref,
                    s_ref, out_ref):
    acc = jnp.dot(ep_lo_ref[...], wb_lo_ref[...],
                  preferred_element_type=jnp.float32)
    acc += jnp.dot(ep_hi_ref[...], wb_hi_ref[...],
                   preferred_element_type=jnp.float32)
    scale = jnp.dot(njp_ref[...], s_ref[...],
                    preferred_element_type=jnp.float32)
    out_ref[...] = acc * scale


def _tc_finish(ep_lo, ep_hi, njp, mean_weight):
    grid = (pl.cdiv(PACK_ROWS, PBLK),)
    eye8 = jnp.eye(8, dtype=jnp.float32)
    wb_lo = jnp.kron(eye8, mean_weight[:HALF, :])
    wb_hi = jnp.kron(eye8, mean_weight[HALF:, :])
    sel = jnp.kron(eye8, jnp.ones((1, EMBED), jnp.float32))
    return pl.pallas_call(
        _tc_finish_body,
        grid=grid,
        in_specs=[
            pl.BlockSpec((PBLK, 128), lambda i: (i, 0)),
            pl.BlockSpec((PBLK, 128), lambda i: (i, 0)),
            pl.BlockSpec((PBLK, 8), lambda i: (i, 0)),
            pl.BlockSpec((128, 256), lambda i: (0, 0)),
            pl.BlockSpec((128, 256), lambda i: (0, 0)),
            pl.BlockSpec((8, 256), lambda i: (0, 0)),
        ],
        out_specs=pl.BlockSpec((PBLK, 256), lambda i: (i, 0)),
        out_shape=jax.ShapeDtypeStruct((PACK_ROWS, 256), jnp.float32),
    )(ep_lo, ep_hi, njp, wb_lo, wb_hi, sel)


def kernel(edge_index, user_n_j, item_n_j, user_emb, item_emb, mean_weight):
    rows1d = edge_index[0].astype(jnp.int32)
    cols1d = edge_index[1].astype(jnp.int32)
    table = item_emb.reshape(2 * N_ITEMS, HALF)
    e_lo, e_hi = _sc_segment_sum(rows1d, cols1d, table)
    ep_lo = e_lo.reshape(PACK_ROWS, 128)
    ep_hi = e_hi.reshape(PACK_ROWS, 128)
    njp = user_n_j.reshape(PACK_ROWS, 8)
    out_pack = _tc_finish(ep_lo, ep_hi, njp, mean_weight)
    return out_pack.reshape(N_USERS, EMBED)


# speedup vs baseline: 30.6950x; 1.2413x over previous
import functools

import jax
import jax.numpy as jnp
from jax import lax
from jax.experimental import pallas as pl
from jax.experimental.pallas import tpu as pltpu
from jax.experimental.pallas import tpu_sc as plsc

N_USERS = 100000
N_ITEMS = 100000
EMBED = 32
HALF = 16

CH = 128
G = 2
NSLOT = 6
AG = 3
AI = NSLOT - 1
ZR = 64
N_EDGE = 1600000
N_CHUNKS = N_EDGE // CH
CHUNKS_PER_TILE = N_CHUNKS // 16
N_MAIN = CHUNKS_PER_TILE - 1
N_GROUPS = N_MAIN // G
ROWS_PER_TILE = 6272
ACC_ROWS = ROWS_PER_TILE * 16
PACK_ROWS = N_USERS // 8


def _sc_segment_sum(edges_il, table):
    mesh = plsc.VectorSubcoreMesh(core_axis_name="c", subcore_axis_name="s")

    @functools.partial(
        pl.kernel,
        out_type=(
            jax.ShapeDtypeStruct((N_USERS, HALF), jnp.float32),
            jax.ShapeDtypeStruct((N_USERS, HALF), jnp.float32),
        ),
        mesh=mesh,
        scratch_types=[
            pltpu.VMEM((NSLOT, G, 2, CH), jnp.int32),
            pltpu.VMEM((NSLOT, G, CH, HALF), jnp.float32),
            pltpu.VMEM((ZR, HALF), jnp.float32),
            pltpu.VMEM_SHARED((ACC_ROWS, HALF), jnp.float32),
            pltpu.SemaphoreType.DMA((NSLOT,)),
            pltpu.SemaphoreType.DMA((NSLOT,)),
            pltpu.SemaphoreType.DMA((NSLOT,)),
        ],
        compiler_params=pltpu.CompilerParams(use_tc_tiling_on_sc=False),
    )
    def seg(edges_hbm, table_hbm, out_lo, out_hi,
            idxb, gath, zbuf, acc, isem, gsem, ssem):
        c = lax.axis_index("c")
        s = lax.axis_index("s")

        def idx_descs(slot, chunk0):
            return [pltpu.make_async_copy(
                edges_hbm.at[pl.ds(chunk0, G)], idxb.at[slot],
                isem.at[slot])]

        def transform_cols(slot):
            for j in range(G):
                for k in range(CH // 16):
                    sl = pl.ds(k * 16, 16)
                    v = idxb[slot, j, 1, sl]
                    idxb[slot, j, 1, sl] = v * 2 + c

        def fire_gathers(slot):
            for j in range(G):
                pltpu.async_copy(table_hbm.at[idxb.at[slot, j, 1]],
                                 gath.at[slot, j], gsem.at[slot])

        def drain_gathers(slot):
            for j in range(G):
                pltpu.make_async_copy(table_hbm.at[idxb.at[slot, j, 1]],
                                      gath.at[slot, j], gsem.at[slot]).wait()

        def fire_scatters(slot):
            for j in range(G):
                pltpu.async_copy(gath.at[slot, j], acc.at[idxb.at[slot, j, 0]],
                                 ssem.at[slot], add=True)

        def drain_scatters(slot):
            for j in range(G):
                pltpu.make_async_copy(gath.at[slot, j],
                                      acc.at[idxb.at[slot, j, 0]],
                                      ssem.at[slot]).wait()

        def run(out):
            chunk_base = s * CHUNKS_PER_TILE
            for q in range(NSLOT):
                for d in idx_descs(q, chunk_base + q * G):
                    d.start()
            for q in range(AG):
                for d in idx_descs(q, chunk_base + q * G):
                    d.wait()
                transform_cols(q)
                fire_gathers(q)

            def zb(i, carry):
                zbuf[i, :] = jnp.zeros((HALF,), jnp.float32)
                return carry

            lax.fori_loop(0, ZR, zb, 0)

            def za(k, carry):
                pltpu.sync_copy(
                    zbuf, acc.at[pl.ds(s * ROWS_PER_TILE + k * ZR, ZR)])
                return carry

            lax.fori_loop(0, ROWS_PER_TILE // ZR, za, 0)
            plsc.subcore_barrier()

            def grp(r, carry):
                p = lax.rem(r, NSLOT)
                pg = lax.rem(r + AG, NSLOT)
                pi = lax.rem(r + AI, NSLOT)
                drain_gathers(p)
                fire_scatters(p)

                @pl.when(r + AG < N_GROUPS)
                def _():
                    for d in idx_descs(pg, chunk_base + (r + AG) * G):
                        d.wait()
                    transform_cols(pg)
                    fire_gathers(pg)

                @pl.when(jnp.logical_and(r >= 1, r + AI < N_GROUPS))
                def _():
                    drain_scatters(pi)
                    for d in idx_descs(pi, chunk_base + (r + AI) * G):
                        d.start()

                return carry

            lax.fori_loop(0, N_GROUPS, grp, 0)
            for g in range(N_GROUPS - NSLOT, N_GROUPS):
                drain_scatters(g % NSLOT)

            def do_chunk(chunk_idx):
                pltpu.sync_copy(edges_hbm.at[pl.ds(chunk_idx, 1)],
                                idxb.at[0, pl.ds(0, 1)])
                for k in range(CH // 16):
                    sl = pl.ds(k * 16, 16)
                    v = idxb[0, 0, 1, sl]
                    idxb[0, 0, 1, sl] = v * 2 + c
                pltpu.async_copy(table_hbm.at[idxb.at[0, 0, 1]],
                                 gath.at[0, 0], gsem.at[0]).wait()
                pltpu.async_copy(gath.at[0, 0], acc.at[idxb.at[0, 0, 0]],
                                 ssem.at[0], add=True).wait()

            do_chunk(chunk_base + N_MAIN)

            @pl.when(s < 4)
            def _():
                do_chunk(16 * CHUNKS_PER_TILE + s)

            plsc.subcore_barrier()

            @pl.when(s < 15)
            def _():
                off = s * ROWS_PER_TILE
                pltpu.sync_copy(acc.at[pl.ds(off, ROWS_PER_TILE)],
                                out.at[pl.ds(off, ROWS_PER_TILE)])

            @pl.when(s == 15)
            def _():
                off = 15 * ROWS_PER_TILE
                rem = N_USERS - off
                pltpu.sync_copy(acc.at[pl.ds(off, rem)],
                                out.at[pl.ds(off, rem)])

        @pl.when(c == 0)
        def _():
            run(out_lo)

        @pl.when(c == 1)
        def _():
            run(out_hi)

    return seg(edges_il, table)


PBLK = 2048
OBLK = PBLK * 8


def _tc_finish_body(ep_lo_ref, ep_hi_ref, njp_ref, wb_lo_ref, wb_hi_ref,
                    s_ref, out_ref):
    acc = jnp.dot(ep_lo_ref[...], wb_lo_ref[...],
                  preferred_element_type=jnp.float32)
    acc += jnp.dot(ep_hi_ref[...], wb_hi_ref[...],
                   preferred_element_type=jnp.float32)
    scale = jnp.dot(njp_ref[...], s_ref[...],
                    preferred_element_type=jnp.float32)
    out_ref[...] = acc * scale


def _tc_finish(ep_lo, ep_hi, njp, mean_weight):
    grid = (pl.cdiv(PACK_ROWS, PBLK),)
    eye8 = jnp.eye(8, dtype=jnp.float32)
    wb_lo = jnp.kron(eye8, mean_weight[:HALF, :])
    wb_hi = jnp.kron(eye8, mean_weight[HALF:, :])
    sel = jnp.kron(eye8, jnp.ones((1, EMBED), jnp.float32))
    return pl.pallas_call(
        _tc_finish_body,
        grid=grid,
        in_specs=[
            pl.BlockSpec((PBLK, 128), lambda i: (i, 0)),
            pl.BlockSpec((PBLK, 128), lambda i: (i, 0)),
            pl.BlockSpec((PBLK, 8), lambda i: (i, 0)),
            pl.BlockSpec((128, 256), lambda i: (0, 0)),
            pl.BlockSpec((128, 256), lambda i: (0, 0)),
            pl.BlockSpec((8, 256), lambda i: (0, 0)),
        ],
        out_specs=pl.BlockSpec((PBLK, 256), lambda i: (i, 0)),
        out_shape=jax.ShapeDtypeStruct((PACK_ROWS, 256), jnp.float32),
    )(ep_lo, ep_hi, njp, wb_lo, wb_hi, sel)


def kernel(edge_index, user_n_j, item_n_j, user_emb, item_emb, mean_weight):
    edges_il = jnp.transpose(
        edge_index.astype(jnp.int32).reshape(2, N_CHUNKS, CH), (1, 0, 2))
    table = item_emb.reshape(2 * N_ITEMS, HALF)
    e_lo, e_hi = _sc_segment_sum(edges_il, table)
    ep_lo = e_lo.reshape(PACK_ROWS, 128)
    ep_hi = e_hi.reshape(PACK_ROWS, 128)
    njp = user_n_j.reshape(PACK_ROWS, 8)
    out_pack = _tc_finish(ep_lo, ep_hi, njp, mean_weight)
    return out_pack.reshape(N_USERS, EMBED)
